# Initial kernel scaffold; baseline (speedup 1.0000x reference)
import jax
import jax.numpy as jnp
from jax.experimental import pallas as pl


def kernel(edge_index, d, edge_h, W1, b1, W2, b2, W3, b3):
    raise NotImplementedError("write your pallas kernel here")


# speedup vs baseline: 1.5014x; 1.5014x over previous
import functools
import math

import jax
import jax.numpy as jnp
from jax import lax
from jax.experimental import pallas as pl
from jax.experimental.pallas import tpu as pltpu
from jax.experimental.pallas import tpu_sc as plsc

N_NODES = 10000
N_EDGES = 320000
FEATS = 128
R_MAX = 5.0
GAP = R_MAX / FEATS
COEFF = -0.5 / (GAP * GAP)
NEG_BIG = -1e30
LOG2 = math.log(2.0)

EB = 4000
NB = 2000

NSUB = 16
TPB = N_EDGES // NSUB
CH = 80
NCH = TPB // CH
RPT = 624


def _k1_body(d_ref, h_ref, mu_ref, out_ref):
    d = d_ref[:]
    eh = h_ref[:]
    mu = mu_ref[:]
    diff = d - mu
    bf = jnp.exp(COEFF * diff * diff)
    c = 0.5 * (jnp.cos(jnp.pi * d / R_MAX) + 1.0)
    c = jnp.where(d < R_MAX, c, 0.0)
    m = eh * bf * (c * c)
    am = jnp.abs(m)
    out_ref[0] = jnp.where(am > 0, jnp.log(am), NEG_BIG)
    out_ref[1] = jnp.where(m < 0, 1.0, 0.0)


def _k1(d2, edge_h, mu):
    return pl.pallas_call(
        _k1_body,
        grid=(N_EDGES // EB,),
        in_specs=[
            pl.BlockSpec((EB, 1), lambda i: (i, 0)),
            pl.BlockSpec((EB, FEATS), lambda i: (i, 0)),
            pl.BlockSpec((1, FEATS), lambda i: (0, 0)),
        ],
        out_specs=pl.BlockSpec((2, EB, FEATS), lambda i: (0, i, 0)),
        out_shape=jax.ShapeDtypeStruct((2, N_EDGES, FEATS), jnp.float32),
    )(d2, edge_h, mu)


def _sc_scatter(F, dst):
    mesh = plsc.VectorSubcoreMesh(core_axis_name="c", subcore_axis_name="s")

    @functools.partial(
        pl.kernel,
        mesh=mesh,
        out_type=jax.ShapeDtypeStruct((2, N_NODES, FEATS), jnp.float32),
        scratch_types=[
            pltpu.VMEM((CH, FEATS), jnp.float32),
            pltpu.VMEM((CH,), jnp.int32),
            pltpu.VMEM_SHARED((N_NODES, FEATS), jnp.float32),
        ],
    )
    def k(f_hbm, dst_hbm, out_hbm, data_v, idx_v, acc_sh):
        c = lax.axis_index("c")
        s = lax.axis_index("s")

        zero16 = jnp.zeros((16,), jnp.float32)

        def _zs(i, carry):
            data_v[i // 8, pl.ds((i % 8) * 16, 16)] = zero16
            return carry

        lax.fori_loop(0, CH * (FEATS // 16), _zs, 0)
        base = s * RPT
        for j in range(7):
            pltpu.sync_copy(data_v.at[pl.ds(0, CH)],
                            acc_sh.at[pl.ds(base + j * CH, CH)])
        last = NSUB - 1

        @pl.when(s == last)
        def _():
            pltpu.sync_copy(data_v.at[pl.ds(0, CH)],
                            acc_sh.at[pl.ds(base + 7 * CH, CH)])

        @pl.when(s != last)
        def _():
            pltpu.sync_copy(data_v.at[pl.ds(0, 64)],
                            acc_sh.at[pl.ds(base + 7 * CH, 64)])

        plsc.subcore_barrier()

        def _chunk(j, carry):
            e0 = s * TPB + j * CH
            pltpu.sync_copy(f_hbm.at[c, pl.ds(e0, CH)], data_v)
            pltpu.sync_copy(dst_hbm.at[pl.ds(e0, CH)], idx_v)
            pltpu.sync_copy(data_v, acc_sh.at[idx_v], add=True)
            return carry

        lax.fori_loop(0, NCH, _chunk, 0)
        plsc.subcore_barrier()

        @pl.when(s == last)
        def _():
            pltpu.sync_copy(acc_sh.at[pl.ds(base, 640)],
                            out_hbm.at[c, pl.ds(base, 640)])

        @pl.when(s != last)
        def _():
            pltpu.sync_copy(acc_sh.at[pl.ds(base, RPT)],
                            out_hbm.at[c, pl.ds(base, RPT)])

    return k(F, dst)


def _k3_body(s_ref, c_ref, w_ref, b_ref, o_ref):
    S = s_ref[:]
    C = c_ref[:]
    odd = C - 2.0 * jnp.floor(C * 0.5)
    h = jnp.exp(S) * (1.0 - 2.0 * odd)
    x = jnp.dot(h, w_ref[:], preferred_element_type=jnp.float32) + b_ref[:]
    o_ref[:] = jnp.maximum(x, 0.0) + jnp.log1p(jnp.exp(-jnp.abs(x))) - LOG2


def _k3(S, C, W3, b3):
    return pl.pallas_call(
        _k3_body,
        grid=(N_NODES // NB,),
        in_specs=[
            pl.BlockSpec((NB, FEATS), lambda i: (i, 0)),
            pl.BlockSpec((NB, FEATS), lambda i: (i, 0)),
            pl.BlockSpec((FEATS, FEATS), lambda i: (0, 0)),
            pl.BlockSpec((1, FEATS), lambda i: (0, 0)),
        ],
        out_specs=pl.BlockSpec((NB, FEATS), lambda i: (i, 0)),
        out_shape=jax.ShapeDtypeStruct((N_NODES, FEATS), jnp.float32),
    )(S, C, W3, b3)


def kernel(edge_index, d, edge_h, W1, b1, W2, b2, W3, b3):
    dst = edge_index[1]
    mu = jnp.linspace(0.0, R_MAX, FEATS, dtype=jnp.float32).reshape(1, FEATS)
    F = _k1(d.reshape(N_EDGES, 1), edge_h, mu)
    sums = _sc_scatter(F, dst)
    return _k3(sums[0], sums[1], W3, b3.reshape(1, FEATS))


# speedup vs baseline: 1.9031x; 1.2676x over previous
import functools
import math

import jax
import jax.numpy as jnp
from jax import lax
from jax.experimental import pallas as pl
from jax.experimental.pallas import tpu as pltpu
from jax.experimental.pallas import tpu_sc as plsc

N_NODES = 10000
N_EDGES = 320000
FEATS = 128
R_MAX = 5.0
GAP = R_MAX / FEATS
COEFF = -0.5 / (GAP * GAP)
NEG_BIG = -1e30
LOG2 = math.log(2.0)
INV_LN2 = 1.0 / math.log(2.0)
COEFF2 = COEFF * INV_LN2
_P = (4.8863580e-08, 1.4426868e+00, -7.2111464e-01, 4.7832355e-01,
      -3.4599602e-01, 2.3923166e-01, -1.3453425e-01, 5.0277509e-02,
      -8.8746967e-03)

EB = 4000
NB = 2000

NSUB = 16
TPB = N_EDGES // NSUB
CH = 80
NCH = TPB // CH
RPT = 624


def _k0_body(d_ref, out_ref):
    d = d_ref[:]
    c = 0.5 * (jnp.cos(jnp.pi * d / R_MAX) + 1.0)
    c = jnp.where(d < R_MAX, c, 0.0)
    out_ref[:] = jnp.where(c > 0, 2.0 * INV_LN2 * jnp.log(c), NEG_BIG)


def _k0(dm):
    return pl.pallas_call(
        _k0_body,
        out_shape=jax.ShapeDtypeStruct(dm.shape, jnp.float32),
    )(dm)


def _k1_body(d_ref, lc2_ref, h_ref, mu_ref, out_ref):
    d = d_ref[:]
    lc2 = lc2_ref[:]
    eh = h_ref[:]
    mu = mu_ref[:]
    diff = d - mu
    lbf2 = COEFF2 * diff * diff
    bits = jax.lax.bitcast_convert_type(eh, jnp.int32) & 0x7FFFFFFF
    ex = jax.lax.shift_right_logical(bits, 23)
    t = jax.lax.bitcast_convert_type((bits & 0x7FFFFF) | 0x3F800000,
                                     jnp.float32) - 1.0
    p = _P[8]
    for k in range(7, -1, -1):
        p = p * t + _P[k]
    lg = (ex.astype(jnp.float32) + p) + (lbf2 + (lc2 - 127.0))
    out_ref[0] = jnp.where(lg < -126.0, NEG_BIG, lg)
    out_ref[1] = jnp.where(eh < 0, 1.0, 0.0)


def _k1(d2, lc2, edge_h, mu):
    return pl.pallas_call(
        _k1_body,
        grid=(N_EDGES // EB,),
        in_specs=[
            pl.BlockSpec((EB, 1), lambda i: (i, 0)),
            pl.BlockSpec((EB, 1), lambda i: (i, 0)),
            pl.BlockSpec((EB, FEATS), lambda i: (i, 0)),
            pl.BlockSpec((1, FEATS), lambda i: (0, 0)),
        ],
        out_specs=pl.BlockSpec((2, EB, FEATS), lambda i: (0, i, 0)),
        out_shape=jax.ShapeDtypeStruct((2, N_EDGES, FEATS), jnp.float32),
    )(d2, lc2, edge_h, mu)


def _sc_scatter(F, dst):
    mesh = plsc.VectorSubcoreMesh(core_axis_name="c", subcore_axis_name="s")

    @functools.partial(
        pl.kernel,
        mesh=mesh,
        out_type=jax.ShapeDtypeStruct((2, N_NODES, FEATS), jnp.float32),
        scratch_types=[
            pltpu.VMEM((CH, FEATS), jnp.float32),
            pltpu.VMEM((CH,), jnp.int32),
            pltpu.VMEM_SHARED((N_NODES, FEATS), jnp.float32),
        ],
    )
    def k(f_hbm, dst_hbm, out_hbm, data_v, idx_v, acc_sh):
        c = lax.axis_index("c")
        s = lax.axis_index("s")

        zero16 = jnp.zeros((16,), jnp.float32)

        def _zs(i, carry):
            data_v[i // 8, pl.ds((i % 8) * 16, 16)] = zero16
            return carry

        lax.fori_loop(0, CH * (FEATS // 16), _zs, 0)
        base = s * RPT
        for j in range(7):
            pltpu.sync_copy(data_v.at[pl.ds(0, CH)],
                            acc_sh.at[pl.ds(base + j * CH, CH)])
        last = NSUB - 1

        @pl.when(s == last)
        def _():
            pltpu.sync_copy(data_v.at[pl.ds(0, CH)],
                            acc_sh.at[pl.ds(base + 7 * CH, CH)])

        @pl.when(s != last)
        def _():
            pltpu.sync_copy(data_v.at[pl.ds(0, 64)],
                            acc_sh.at[pl.ds(base + 7 * CH, 64)])

        plsc.subcore_barrier()

        def _chunk(j, carry):
            e0 = s * TPB + j * CH
            pltpu.sync_copy(f_hbm.at[c, pl.ds(e0, CH)], data_v)
            pltpu.sync_copy(dst_hbm.at[pl.ds(e0, CH)], idx_v)
            pltpu.sync_copy(data_v, acc_sh.at[idx_v], add=True)
            return carry

        lax.fori_loop(0, NCH, _chunk, 0)
        plsc.subcore_barrier()

        @pl.when(s == last)
        def _():
            pltpu.sync_copy(acc_sh.at[pl.ds(base, 640)],
                            out_hbm.at[c, pl.ds(base, 640)])

        @pl.when(s != last)
        def _():
            pltpu.sync_copy(acc_sh.at[pl.ds(base, RPT)],
                            out_hbm.at[c, pl.ds(base, RPT)])

    return k(F, dst)


def _k3_body(s_ref, c_ref, w_ref, b_ref, o_ref):
    S = s_ref[:]
    C = c_ref[:]
    odd = C - 2.0 * jnp.floor(C * 0.5)
    h = jnp.exp2(S) * (1.0 - 2.0 * odd)
    x = jnp.dot(h, w_ref[:], preferred_element_type=jnp.float32) + b_ref[:]
    o_ref[:] = jnp.maximum(x, 0.0) + jnp.log1p(jnp.exp(-jnp.abs(x))) - LOG2


def _k3(S, C, W3, b3):
    return pl.pallas_call(
        _k3_body,
        grid=(N_NODES // NB,),
        in_specs=[
            pl.BlockSpec((NB, FEATS), lambda i: (i, 0)),
            pl.BlockSpec((NB, FEATS), lambda i: (i, 0)),
            pl.BlockSpec((FEATS, FEATS), lambda i: (0, 0)),
            pl.BlockSpec((1, FEATS), lambda i: (0, 0)),
        ],
        out_specs=pl.BlockSpec((NB, FEATS), lambda i: (i, 0)),
        out_shape=jax.ShapeDtypeStruct((N_NODES, FEATS), jnp.float32),
    )(S, C, W3, b3)


def kernel(edge_index, d, edge_h, W1, b1, W2, b2, W3, b3):
    dst = edge_index[1]
    mu = jnp.linspace(0.0, R_MAX, FEATS, dtype=jnp.float32).reshape(1, FEATS)
    lc2 = _k0(d.reshape(N_EDGES // FEATS, FEATS)).reshape(N_EDGES, 1)
    F = _k1(d.reshape(N_EDGES, 1), lc2, edge_h, mu)
    sums = _sc_scatter(F, dst)
    return _k3(sums[0], sums[1], W3, b3.reshape(1, FEATS))


# speedup vs baseline: 2.7033x; 1.4205x over previous
import functools
import math

import jax
import jax.numpy as jnp
from jax import lax
from jax.experimental import pallas as pl
from jax.experimental.pallas import tpu as pltpu
from jax.experimental.pallas import tpu_sc as plsc

N_NODES = 10000
N_EDGES = 320000
FEATS = 128
R_MAX = 5.0
GAP = R_MAX / FEATS
COEFF = -0.5 / (GAP * GAP)
NEG_BIG = -1e30
LOG2 = math.log(2.0)
INV_LN2 = 1.0 / math.log(2.0)
COEFF2 = COEFF * INV_LN2
_P = (4.8863580e-08, 1.4426868e+00, -7.2111464e-01, 4.7832355e-01,
      -3.4599602e-01, 2.3923166e-01, -1.3453425e-01, 5.0277509e-02,
      -8.8746967e-03)

EB = 4000
NB = 2000

NSUB = 16
TPB = N_EDGES // NSUB
SB = 80
CH = 160
NCH = TPB // CH
RPT = 624


def _k0_body(d_ref, out_ref):
    d = d_ref[:]
    c = 0.5 * (jnp.cos(jnp.pi * d / R_MAX) + 1.0)
    c = jnp.where(d < R_MAX, c, 0.0)
    out_ref[:] = jnp.where(c > 0, 2.0 * INV_LN2 * jnp.log(c), NEG_BIG)


def _k0(dm):
    return pl.pallas_call(
        _k0_body,
        out_shape=jax.ShapeDtypeStruct(dm.shape, jnp.float32),
    )(dm)


def _k1_body(d_ref, lc2_ref, h_ref, mu_ref, out_ref):
    d = d_ref[:]
    lc2 = lc2_ref[:]
    eh = h_ref[:]
    mu = mu_ref[:]
    diff = d - mu
    lbf2 = COEFF2 * diff * diff
    bits = jax.lax.bitcast_convert_type(eh, jnp.int32) & 0x7FFFFFFF
    ex = jax.lax.shift_right_logical(bits, 23)
    t = jax.lax.bitcast_convert_type((bits & 0x7FFFFF) | 0x3F800000,
                                     jnp.float32) - 1.0
    p = _P[8]
    for k in range(7, -1, -1):
        p = p * t + _P[k]
    lg = (ex.astype(jnp.float32) + p) + (lbf2 + (lc2 - 127.0))
    out_ref[0] = jnp.where(lg < -126.0, NEG_BIG, lg)
    out_ref[1] = jnp.where(eh < 0, 1.0, 0.0)


def _k1(d2, lc2, edge_h, mu):
    return pl.pallas_call(
        _k1_body,
        grid=(N_EDGES // EB,),
        in_specs=[
            pl.BlockSpec((EB, 1), lambda i: (i, 0)),
            pl.BlockSpec((EB, 1), lambda i: (i, 0)),
            pl.BlockSpec((EB, FEATS), lambda i: (i, 0)),
            pl.BlockSpec((1, FEATS), lambda i: (0, 0)),
        ],
        out_specs=pl.BlockSpec((2, EB, FEATS), lambda i: (0, i, 0)),
        out_shape=jax.ShapeDtypeStruct((2, N_EDGES, FEATS), jnp.float32),
    )(d2, lc2, edge_h, mu)


def _sc_scatter(F, dst3):
    mesh = plsc.VectorSubcoreMesh(core_axis_name="c", subcore_axis_name="s")

    @functools.partial(
        pl.kernel,
        mesh=mesh,
        out_type=jax.ShapeDtypeStruct((2, N_NODES, FEATS), jnp.float32),
        scratch_types=[
            pltpu.VMEM((CH, FEATS), jnp.float32),
            pltpu.VMEM((CH, FEATS), jnp.float32),
            pltpu.VMEM((CH // SB, SB), jnp.int32),
            pltpu.VMEM((CH // SB, SB), jnp.int32),
            pltpu.VMEM_SHARED((N_NODES, FEATS), jnp.float32),
            pltpu.SemaphoreType.DMA,
            pltpu.SemaphoreType.DMA,
            pltpu.SemaphoreType.DMA,
            pltpu.SemaphoreType.DMA,
        ],
    )
    def k(f_hbm, dst_hbm, out_hbm, data0, data1, idx0, idx1, acc_sh,
          sd0, sd1, si0, si1):
        c = lax.axis_index("c")
        s = lax.axis_index("s")
        datas, idxs = (data0, data1), (idx0, idx1)
        sds, sis = (sd0, sd1), (si0, si1)

        zero16 = jnp.zeros((16,), jnp.float32)

        def _zs(i, carry):
            data0[i // 8, pl.ds((i % 8) * 16, 16)] = zero16
            return carry

        lax.fori_loop(0, 80 * (FEATS // 16), _zs, 0)
        base = s * RPT
        for j in range(7):
            pltpu.sync_copy(data0.at[pl.ds(0, 80)],
                            acc_sh.at[pl.ds(base + j * 80, 80)])
        last = NSUB - 1

        @pl.when(s == last)
        def _():
            pltpu.sync_copy(data0.at[pl.ds(0, 80)],
                            acc_sh.at[pl.ds(base + 7 * 80, 80)])

        @pl.when(s != last)
        def _():
            pltpu.sync_copy(data0.at[pl.ds(0, 64)],
                            acc_sh.at[pl.ds(base + 7 * 80, 64)])

        plsc.subcore_barrier()

        def _start(j, b):
            e0 = s * TPB + j * CH
            pltpu.async_copy(f_hbm.at[c, pl.ds(e0, CH)], datas[b], sds[b])
            pltpu.async_copy(dst_hbm.at[s * NCH + j], idxs[b], sis[b])

        def _wait(b):
            pltpu.make_async_copy(f_hbm.at[0, pl.ds(0, CH)],
                                  datas[b], sds[b]).wait()
            pltpu.make_async_copy(dst_hbm.at[0], idxs[b], sis[b]).wait()

        _start(0, 0)
        _start(1, 1)

        def _scat(b):
            for q in range(CH // SB):
                pltpu.sync_copy(datas[b].at[pl.ds(q * SB, SB)],
                                acc_sh.at[idxs[b].at[q]], add=True)

        def _pair(i, carry):
            for b in range(2):
                j = 2 * i + b
                _wait(b)
                _scat(b)

                @pl.when(j + 2 < NCH)
                def _():
                    _start(j + 2, b)
            return carry

        lax.fori_loop(0, NCH // 2, _pair, 0)
        if NCH % 2:
            _wait(0)
            _scat(0)
        plsc.subcore_barrier()

        @pl.when(s == last)
        def _():
            pltpu.sync_copy(acc_sh.at[pl.ds(base, 640)],
                            out_hbm.at[c, pl.ds(base, 640)])

        @pl.when(s != last)
        def _():
            pltpu.sync_copy(acc_sh.at[pl.ds(base, RPT)],
                            out_hbm.at[c, pl.ds(base, RPT)])

    return k(F, dst3)


def _k3_body(s_ref, c_ref, w_ref, b_ref, o_ref):
    S = s_ref[:]
    C = c_ref[:]
    odd = C - 2.0 * jnp.floor(C * 0.5)
    h = jnp.exp2(S) * (1.0 - 2.0 * odd)
    x = jnp.dot(h, w_ref[:], preferred_element_type=jnp.float32) + b_ref[:]
    o_ref[:] = jnp.maximum(x, 0.0) + jnp.log1p(jnp.exp(-jnp.abs(x))) - LOG2


def _k3(S, C, W3, b3):
    return pl.pallas_call(
        _k3_body,
        grid=(N_NODES // NB,),
        in_specs=[
            pl.BlockSpec((NB, FEATS), lambda i: (i, 0)),
            pl.BlockSpec((NB, FEATS), lambda i: (i, 0)),
            pl.BlockSpec((FEATS, FEATS), lambda i: (0, 0)),
            pl.BlockSpec((1, FEATS), lambda i: (0, 0)),
        ],
        out_specs=pl.BlockSpec((NB, FEATS), lambda i: (i, 0)),
        out_shape=jax.ShapeDtypeStruct((N_NODES, FEATS), jnp.float32),
    )(S, C, W3, b3)


def kernel(edge_index, d, edge_h, W1, b1, W2, b2, W3, b3):
    dst = edge_index[1]
    mu = jnp.linspace(0.0, R_MAX, FEATS, dtype=jnp.float32).reshape(1, FEATS)
    lc2 = _k0(d.reshape(N_EDGES // FEATS, FEATS)).reshape(N_EDGES, 1)
    F = _k1(d.reshape(N_EDGES, 1), lc2, edge_h, mu)
    sums = _sc_scatter(F, dst.reshape(N_EDGES // CH, CH // SB, SB))
    return _k3(sums[0], sums[1], W3, b3.reshape(1, FEATS))
